# baseline (speedup 1.0000x reference)
import functools

import jax
import jax.numpy as jnp
from jax import lax
from jax.experimental import pallas as pl
from jax.experimental.pallas import tpu as pltpu
from jax.experimental.pallas import tpu_sc as plsc

N = 100000
B = 256
D = 128
W = 64
BN = 4000
NBLK = N // BN
NEG = -1e30

NTILES = 32
CH = 3200
NPAD = NTILES * CH
PADID = 300
HB = 320


def _leaky(x):
    return jnp.where(x >= 0, x, 0.1 * x)


def _hist_body(ids_hbm, out_hbm, ids_v, hist_v, sum_v, shared_v):
    c = lax.axis_index("c")
    s = lax.axis_index("s")
    wid = s * 2 + c
    pltpu.sync_copy(ids_hbm.at[pl.ds(wid * CH, CH)], ids_v)
    zero = jnp.zeros((16,), jnp.float32)
    for k in range(16 * HB // 16):
        hist_v[pl.ds(k * 16, 16)] = zero
    laneoff = lax.broadcasted_iota(jnp.int32, (16,), 0) * HB
    ones = jnp.ones((16,), jnp.float32)

    def step(k, carry):
        idx = ids_v[pl.ds(k * 16, 16)] + laneoff
        plsc.addupdate_scatter(hist_v, [idx], ones)
        return carry

    lax.fori_loop(0, CH // 16, step, 0)
    for k in range(HB // 16):
        acc = hist_v[pl.ds(k * 16, 16)]
        for r in range(1, 16):
            acc = acc + hist_v[pl.ds(r * HB + k * 16, 16)]
        sum_v[pl.ds(k * 16, 16)] = acc
    pltpu.sync_copy(sum_v, shared_v.at[pl.ds(s * HB, HB)])
    plsc.subcore_barrier()

    @pl.when(s == 0)
    def _():
        pltpu.sync_copy(shared_v, hist_v)
        for k in range(B // 16):
            acc = hist_v[pl.ds(k * 16, 16)]
            for r in range(1, 16):
                acc = acc + hist_v[pl.ds(r * HB + k * 16, 16)]
            sum_v[pl.ds(k * 16, 16)] = acc
        pltpu.sync_copy(sum_v.at[pl.ds(0, B)], out_hbm.at[c])


def _sc_counts(ids_pad, interpret=False):
    mesh = plsc.VectorSubcoreMesh(core_axis_name="c", subcore_axis_name="s")
    return pl.kernel(
        _hist_body,
        out_type=jax.ShapeDtypeStruct((2, B), jnp.float32),
        mesh=mesh,
        scratch_types=[
            pltpu.VMEM((CH,), jnp.int32),
            pltpu.VMEM((16 * HB,), jnp.float32),
            pltpu.VMEM((HB,), jnp.float32),
            pltpu.VMEM_SHARED((16 * HB,), jnp.float32),
        ],
        compiler_params=pltpu.CompilerParams(needs_layout_passes=False),
        interpret=interpret,
    )(ids_pad)


def _main_body(wf_ref, ids_ref, feats_ref, w1_ref, w2_ref, ones_ref, cnt2_ref,
               out_ref, tmax_ref, acc_ref, den_ref):
    i = pl.program_id(0)
    o1 = lax.dot_general(wf_ref[...], w1_ref[...], (((1,), (1,)), ((), ())),
                         preferred_element_type=jnp.float32)
    t = lax.dot_general(w2_ref[...], o1, (((1,), (1,)), ((), ())),
                        preferred_element_type=jnp.float32)
    t = _leaky(t)
    m_old = jnp.where(i == 0, NEG, tmax_ref[0])
    m_new = jnp.maximum(m_old, jnp.max(t))
    tmax_ref[0] = m_new
    factor = jnp.exp(m_old - m_new)
    e = jnp.exp(t - m_new)
    ids = ids_ref[0, 0, :].reshape(1, BN)
    oh = lax.broadcasted_iota(jnp.int32, (B, BN), 0) == ids
    w_bf = jnp.where(oh, e, 0.0).astype(jnp.bfloat16)
    bacc = lax.dot_general(w_bf, feats_ref[...].astype(jnp.bfloat16),
                           (((1,), (0,)), ((), ())),
                           preferred_element_type=jnp.float32)
    bden = lax.dot_general(w_bf, ones_ref[...], (((1,), (0,)), ((), ())),
                           preferred_element_type=jnp.float32)[:, 0:1]
    first = i == 0
    acc_ref[...] = jnp.where(first, bacc, acc_ref[...] * factor + bacc)
    den_ref[...] = jnp.where(first, bden, den_ref[...] * factor + bden)

    @pl.when(i == NBLK - 1)
    def _():
        cnt = lax.dot_general(cnt2_ref[...], jnp.ones((2, 1), jnp.float32),
                              (((0,), (0,)), ((), ())),
                              preferred_element_type=jnp.float32)
        mean_nodes = float(N) / float(B)
        scale = mean_nodes / (jnp.maximum(den_ref[...], 1e-30)
                              * jnp.maximum(cnt, 1.0))
        out_ref[...] = _leaky(acc_ref[...] * scale)


@functools.partial(jax.jit, static_argnames=("interpret",))
def kernel(node_feats, weights_feats, segment_ids, W1, W2, interpret=False):
    ids32 = segment_ids.astype(jnp.int32)
    ids_pad = jnp.concatenate(
        [ids32, jnp.full((NPAD - N,), PADID, jnp.int32)])
    cnt2 = _sc_counts(ids_pad, interpret=interpret)

    ids3 = ids32.reshape(NBLK, 1, BN)
    ones_bf = jnp.ones((BN, D), jnp.bfloat16)

    out = pl.pallas_call(
        _main_body,
        grid=(NBLK,),
        in_specs=[
            pl.BlockSpec((BN, W), lambda i: (i, 0)),
            pl.BlockSpec((1, 1, BN), lambda i: (i, 0, 0)),
            pl.BlockSpec((BN, D), lambda i: (i, 0)),
            pl.BlockSpec((2 * W, W), lambda i: (0, 0)),
            pl.BlockSpec((1, 2 * W), lambda i: (0, 0)),
            pl.BlockSpec((BN, D), lambda i: (0, 0)),
            pl.BlockSpec((2, B), lambda i: (0, 0)),
        ],
        out_specs=pl.BlockSpec((B, D), lambda i: (0, 0)),
        out_shape=jax.ShapeDtypeStruct((B, D), jnp.float32),
        scratch_shapes=[
            pltpu.SMEM((1,), jnp.float32),
            pltpu.VMEM((B, D), jnp.float32),
            pltpu.VMEM((B, 1), jnp.float32),
        ],
        interpret=interpret,
    )(weights_feats, ids3, node_feats, W1, W2, ones_bf, cnt2)

    return out


# speedup vs baseline: 2.2518x; 2.2518x over previous
import functools

import jax
import jax.numpy as jnp
from jax import lax
from jax.experimental import pallas as pl
from jax.experimental.pallas import tpu as pltpu
from jax.experimental.pallas import tpu_sc as plsc

N = 100000
B = 256
D = 128
W = 64
BN = 3200
NBLK = 32
NEG = -1e30

NTILES = 32
CH = 3200
NPAD = NTILES * CH
PADID = 300
HB = 320


def _leaky(x):
    return jnp.where(x >= 0, x, 0.1 * x)


def _hist_body(ids_hbm, out_hbm, ids_v, hist_v, sum_v, shared_v):
    c = lax.axis_index("c")
    s = lax.axis_index("s")
    wid = s * 2 + c
    pltpu.sync_copy(ids_hbm.at[pl.ds(wid * CH, CH)], ids_v)
    zero = jnp.zeros((16,), jnp.float32)
    for k in range(16 * HB // 16):
        hist_v[pl.ds(k * 16, 16)] = zero
    laneoff = lax.broadcasted_iota(jnp.int32, (16,), 0) * HB
    ones = jnp.ones((16,), jnp.float32)

    def step(k, carry):
        idx = ids_v[pl.ds(k * 16, 16)] + laneoff
        plsc.addupdate_scatter(hist_v, [idx], ones)
        return carry

    lax.fori_loop(0, CH // 16, step, 0)
    for k in range(HB // 16):
        acc = hist_v[pl.ds(k * 16, 16)]
        for r in range(1, 16):
            acc = acc + hist_v[pl.ds(r * HB + k * 16, 16)]
        sum_v[pl.ds(k * 16, 16)] = acc
    pltpu.sync_copy(sum_v, shared_v.at[pl.ds(s * HB, HB)])
    plsc.subcore_barrier()

    @pl.when(s == 0)
    def _():
        pltpu.sync_copy(shared_v, hist_v)
        for k in range(B // 16):
            acc = hist_v[pl.ds(k * 16, 16)]
            for r in range(1, 16):
                acc = acc + hist_v[pl.ds(r * HB + k * 16, 16)]
            sum_v[pl.ds(k * 16, 16)] = acc
        pltpu.sync_copy(sum_v.at[pl.ds(0, B)], out_hbm.at[c])


def _sc_counts(ids_pad, interpret=False):
    mesh = plsc.VectorSubcoreMesh(core_axis_name="c", subcore_axis_name="s")
    return pl.kernel(
        _hist_body,
        out_type=jax.ShapeDtypeStruct((2, B), jnp.float32),
        mesh=mesh,
        scratch_types=[
            pltpu.VMEM((CH,), jnp.int32),
            pltpu.VMEM((16 * HB,), jnp.float32),
            pltpu.VMEM((HB,), jnp.float32),
            pltpu.VMEM_SHARED((16 * HB,), jnp.float32),
        ],
        compiler_params=pltpu.CompilerParams(needs_layout_passes=False),
        interpret=interpret,
    )(ids_pad)


def _main_body(wft_ref, ids_ref, feats_ref, w1t_ref, w2_ref, ones_ref, cnt2_ref,
               out_ref, tmp_ref, tmax_ref, acc_ref, den_ref):
    i = pl.program_id(0)

    @pl.when(i == 0)
    def _scores():
        v = lax.dot_general(w2_ref[...], w1t_ref[...], (((1,), (1,)), ((), ())),
                            preferred_element_type=jnp.float32)
        t_all = lax.dot_general(v, wft_ref[...], (((1,), (0,)), ((), ())),
                                preferred_element_type=jnp.float32)
        t_all = _leaky(t_all)
        tmp_ref[0:1, pl.ds(0, N)] = t_all
        tmp_ref[0:1, pl.ds(N, NPAD - N)] = jnp.full((1, NPAD - N), NEG,
                                                    jnp.float32)
        tmax_ref[0] = jnp.max(t_all)

    ids = ids_ref[0, 0, :].reshape(1, BN)
    oh = lax.broadcasted_iota(jnp.int32, (B, BN), 0) == ids
    t = tmp_ref[0:1, pl.ds(i * BN, BN)]
    e = jnp.exp(t - tmax_ref[0])
    w_bf = jnp.where(oh, e, 0.0).astype(jnp.bfloat16)
    rowvalid = (lax.broadcasted_iota(jnp.int32, (BN, D), 0) + i * BN) < N
    f_bf = jnp.where(rowvalid, feats_ref[...], 0.0).astype(jnp.bfloat16)
    bacc = lax.dot_general(w_bf, f_bf,
                           (((1,), (0,)), ((), ())),
                           preferred_element_type=jnp.float32)
    bden = lax.dot_general(w_bf, ones_ref[...], (((1,), (0,)), ((), ())),
                           preferred_element_type=jnp.float32)[:, 0:1]
    first = i == 0
    acc_ref[...] = jnp.where(first, 0.0, acc_ref[...]) + bacc
    den_ref[...] = jnp.where(first, 0.0, den_ref[...]) + bden

    @pl.when(i == NBLK - 1)
    def _():
        cnt = lax.dot_general(cnt2_ref[...], jnp.ones((2, 1), jnp.float32),
                              (((0,), (0,)), ((), ())),
                              preferred_element_type=jnp.float32)
        mean_nodes = float(N) / float(B)
        scale = mean_nodes / (jnp.maximum(den_ref[...], 1e-30)
                              * jnp.maximum(cnt, 1.0))
        out_ref[...] = _leaky(acc_ref[...] * scale)


@functools.partial(jax.jit, static_argnames=("interpret",))
def kernel(node_feats, weights_feats, segment_ids, W1, W2, interpret=False):
    ids32 = segment_ids.astype(jnp.int32)
    ids_pad = jnp.concatenate(
        [ids32, jnp.full((NPAD - N,), PADID, jnp.int32)])
    cnt2 = _sc_counts(ids_pad, interpret=interpret)

    ids3 = ids_pad.reshape(NBLK, 1, BN)
    ones_bf = jnp.ones((BN, D), jnp.bfloat16)

    out = pl.pallas_call(
        _main_body,
        grid=(NBLK,),
        in_specs=[
            pl.BlockSpec((W, N), lambda i: (0, 0)),
            pl.BlockSpec((1, 1, BN), lambda i: (i, 0, 0)),
            pl.BlockSpec((BN, D), lambda i: (i, 0)),
            pl.BlockSpec((W, 2 * W), lambda i: (0, 0)),
            pl.BlockSpec((1, 2 * W), lambda i: (0, 0)),
            pl.BlockSpec((BN, D), lambda i: (0, 0)),
            pl.BlockSpec((2, B), lambda i: (0, 0)),
        ],
        out_specs=pl.BlockSpec((B, D), lambda i: (0, 0)),
        out_shape=jax.ShapeDtypeStruct((B, D), jnp.float32),
        scratch_shapes=[
            pltpu.VMEM((1, NPAD), jnp.float32),
            pltpu.SMEM((1,), jnp.float32),
            pltpu.VMEM((B, D), jnp.float32),
            pltpu.VMEM((B, 1), jnp.float32),
        ],
        compiler_params=pltpu.CompilerParams(
            fuse_transposed_lhs_in_matmul=True),
        interpret=interpret,
    )(weights_feats.T, ids3, node_feats, W1.T, W2, ones_bf, cnt2)

    return out
